# Initial kernel scaffold; baseline (speedup 1.0000x reference)
import jax
import jax.numpy as jnp
from jax.experimental import pallas as pl


def kernel(inputs, s0, s1, enc0_Wg, enc0_bg, enc0_Wc, enc0_bc, enc1_Wg, enc1_bg, enc1_Wc, enc1_bc, dec0_Wg, dec0_bg, dec0_Wc, dec0_bc, dec1_Wg, dec1_bg, dec1_Wc, dec1_bc, proj_W, proj_b):
    raise NotImplementedError("write your pallas kernel here")


# speedup vs baseline: 2.0759x; 2.0759x over previous
import jax
import jax.numpy as jnp
from jax import lax
from jax.experimental import pallas as pl
from jax.experimental.pallas import tpu as pltpu

_B = 64
_N = 325
_IN = 2
_OUT = 1
_H = 64
_ORDER = 2
_HORIZON = 12
_SEQ = 12
_K = 5


def _dcrnn_body(xin_ref, s0_ref, s1_ref,
                e0Wg, e0bg, e0Wc, e0bc,
                e1Wg, e1bg, e1Wc, e1bc,
                d0Wg, d0bg, d0Wc, d0bc,
                d1Wg, d1bg, d1Wc, d1bc,
                pWc, pWr, pb, out_ref):
    s0 = s0_ref[...]
    s1 = s1_ref[...]

    def gconv(cat, W_ref, b_ref):
        acc = jnp.dot(cat, W_ref[0], preferred_element_type=jnp.float32)
        acc = acc + b_ref[...]
        k = 1
        for s in (s0, s1):
            xk = cat
            for _ in range(_ORDER):
                xk = jnp.dot(s, xk, preferred_element_type=jnp.float32)
                acc = acc + jnp.dot(xk, W_ref[k],
                                    preferred_element_type=jnp.float32)
                k += 1
        return acc

    def cell(x, h, Wg, bg, Wc, bc):
        ru = jax.nn.sigmoid(gconv(jnp.concatenate([x, h], axis=1), Wg, bg))
        r = ru[:, :_H]
        u = ru[:, _H:]
        c = jnp.tanh(gconv(jnp.concatenate([x, r * h], axis=1), Wc, bc))
        return u * h + (1.0 - u) * c

    z = jnp.zeros((_N, _H), dtype=jnp.float32)

    def enc_step(t, hs):
        h0, h1 = hs
        x = xin_ref[0, t]
        h0 = cell(x, h0, e0Wg, e0bg, e0Wc, e0bc)
        h1 = cell(h0, h1, e1Wg, e1bg, e1Wc, e1bc)
        return (h0, h1)

    h0, h1 = lax.fori_loop(0, _SEQ, enc_step, (z, z))

    def dec_step(t, carry):
        h0, h1, x = carry
        h0 = cell(x, h0, d0Wg, d0bg, d0Wc, d0bc)
        h1 = cell(h0, h1, d1Wg, d1bg, d1Wc, d1bc)
        y_col = jnp.dot(h1, pWc[...], preferred_element_type=jnp.float32)
        y_col = y_col + pb[...]
        y_row = lax.dot_general(pWr[...], h1, (((1,), (1,)), ((), ())),
                                preferred_element_type=jnp.float32)
        out_ref[0, t] = y_row + pb[...].T
        return (h0, h1, y_col)

    x0 = jnp.zeros((_N, _OUT), dtype=jnp.float32)
    lax.fori_loop(0, _HORIZON, dec_step, (h0, h1, x0))


def _perm(W):
    C = W.shape[0] // _K
    return W.reshape(C, _K, W.shape[1]).transpose(1, 0, 2)


def kernel(inputs, s0, s1,
           enc0_Wg, enc0_bg, enc0_Wc, enc0_bc,
           enc1_Wg, enc1_bg, enc1_Wc, enc1_bc,
           dec0_Wg, dec0_bg, dec0_Wc, dec0_bc,
           dec1_Wg, dec1_bg, dec1_Wc, dec1_bc,
           proj_W, proj_b):
    xin = inputs.transpose(0, 3, 2, 1)
    Ws = []
    for Wg, bg, Wc, bc in ((enc0_Wg, enc0_bg, enc0_Wc, enc0_bc),
                           (enc1_Wg, enc1_bg, enc1_Wc, enc1_bc),
                           (dec0_Wg, dec0_bg, dec0_Wc, dec0_bc),
                           (dec1_Wg, dec1_bg, dec1_Wc, dec1_bc)):
        Ws += [_perm(Wg), bg.reshape(1, -1), _perm(Wc), bc.reshape(1, -1)]
    pWc = proj_W.T
    pWr = proj_W
    pb = proj_b.reshape(1, -1)

    def w_spec(a):
        return pl.BlockSpec(a.shape, lambda i: (0,) * a.ndim)

    operands = [xin, s0, s1] + Ws + [pWc, pWr, pb]
    in_specs = [pl.BlockSpec((1, _SEQ, _N, _IN), lambda i: (i, 0, 0, 0))]
    in_specs += [w_spec(a) for a in operands[1:]]

    out = pl.pallas_call(
        _dcrnn_body,
        grid=(_B,),
        in_specs=in_specs,
        out_specs=pl.BlockSpec((1, _HORIZON, _OUT, _N),
                               lambda i: (i, 0, 0, 0)),
        out_shape=jax.ShapeDtypeStruct((_B, _HORIZON, _OUT, _N), jnp.float32),
        compiler_params=pltpu.CompilerParams(
            dimension_semantics=("parallel",)),
    )(*operands)
    return out.transpose(0, 2, 3, 1)
